# Initial kernel scaffold; baseline (speedup 1.0000x reference)
import jax
import jax.numpy as jnp
from jax.experimental import pallas as pl


def kernel(single_index, numerical_index, numerical_value, value, W1s, W1n, W2s, W2n, D1_w, D1_b, bn1_g, bn1_b, bn1_m, bn1_v, D2_w, D2_b, bn2_g, bn2_b, bn2_m, bn2_v, Wf, bf):
    raise NotImplementedError("write your pallas kernel here")


# speedup vs baseline: 25.7592x; 25.7592x over previous
import jax
import jax.numpy as jnp
from jax import lax
from jax.experimental import pallas as pl
from jax.experimental.pallas import tpu as pltpu
from jax.experimental.pallas import tpu_sc as plsc

B = 16384
SINGLE = 26
NUM = 13
EMB = 16
VOCAB = 10000
FIELD = SINGLE + NUM
FP = FIELD + 1
NIDX = B * FP
EPS = 1e-3

NC = 2
NS = 16
NW = NC * NS
G = 128
ROWS = NIDX // G
PER_W_ROWS = ROWS // NW
CG = 8
NCHUNK = PER_W_ROWS // CG
NPAIR = NCHUNK // 2
CR = CG * G
W1N = 2 * VOCAB


def _sc_body(idx_hbm, w2_hbm, w1_hbm, val_hbm, emb_out, yf_out,
             w1_v, idx_v0, idx_v1, rows_v0, rows_v1, val_v0, val_v1,
             yf_v0, yf_v1, in_s0, in_s1, g_s0, g_s1, out_s0, out_s1):
    wid = lax.axis_index("s") * NC + lax.axis_index("c")
    row_base = wid * PER_W_ROWS
    pltpu.sync_copy(w1_hbm, w1_v)

    def start_in(c, sl_idx, sl_val, sem):
        ro = row_base + c * CG
        pltpu.async_copy(idx_hbm.at[pl.ds(ro, CG)], sl_idx, sem)
        pltpu.async_copy(val_hbm.at[pl.ds(ro, CG)], sl_val, sem)

    def wait_in(sl_idx, sl_val, sem):
        pltpu.make_async_copy(idx_hbm.at[pl.ds(0, CG)], sl_idx, sem).wait()
        pltpu.make_async_copy(val_hbm.at[pl.ds(0, CG)], sl_val, sem).wait()

    def fire_g(sl_idx, sl_rows, sem):
        for g in range(CG):
            pltpu.async_copy(w2_hbm.at[sl_idx.at[g]],
                             sl_rows.at[pl.ds(g * G, G)], sem)

    def wait_g(sl_rows, sem):
        pltpu.make_async_copy(w2_hbm.at[pl.ds(0, CR)], sl_rows, sem).wait()

    def start_out(c, sl_rows, sl_yf, sem):
        ro = row_base + c * CG
        pltpu.async_copy(sl_rows, emb_out.at[pl.ds(ro * G, CR)], sem)
        pltpu.async_copy(sl_yf, yf_out.at[pl.ds(ro, CG)], sem)

    def wait_out(sl_rows, sl_yf, sem):
        pltpu.make_async_copy(sl_rows, emb_out.at[pl.ds(0, CR)], sem).wait()
        pltpu.make_async_copy(sl_yf, yf_out.at[pl.ds(0, CG)], sem).wait()

    def first_order(sl_idx, sl_val, sl_yf):
        def body(g, c):
            for k in range(G // 16):
                iv = sl_idx[g, pl.ds(k * 16, 16)]
                vals = plsc.load_gather(w1_v, [iv])
                sl_yf[g, pl.ds(k * 16, 16)] = (
                    vals * sl_val[g, pl.ds(k * 16, 16)])
            return c
        lax.fori_loop(0, CG, body, 0)

    slots = ((idx_v0, val_v0, rows_v0, yf_v0, in_s0, g_s0, out_s0),
             (idx_v1, val_v1, rows_v1, yf_v1, in_s1, g_s1, out_s1))

    start_in(0, idx_v0, val_v0, in_s0)
    start_in(1, idx_v1, val_v1, in_s1)

    def pair(j, carry):
        for s in range(2):
            sl_idx, sl_val, sl_rows, sl_yf, in_s, g_s, out_s = slots[s]
            c = 2 * j + s

            @pl.when(j > 0)
            def _():
                wait_out(sl_rows, sl_yf, out_s)

            wait_in(sl_idx, sl_val, in_s)
            fire_g(sl_idx, sl_rows, g_s)
            first_order(sl_idx, sl_val, sl_yf)
            wait_g(sl_rows, g_s)
            start_out(c, sl_rows, sl_yf, out_s)

            @pl.when(j < NPAIR - 1)
            def _():
                start_in(c + 2, sl_idx, sl_val, in_s)
        return carry

    lax.fori_loop(0, NPAIR, pair, 0)
    wait_out(rows_v0, yf_v0, out_s0)
    wait_out(rows_v1, yf_v1, out_s1)


def _sc_gather(idx2, w2, w1, val2):
    wrapped = pl.kernel(
        _sc_body,
        out_type=(
            jax.ShapeDtypeStruct((NIDX, EMB), jnp.float32),
            jax.ShapeDtypeStruct((ROWS, G), jnp.float32),
        ),
        mesh=plsc.VectorSubcoreMesh(core_axis_name="c", subcore_axis_name="s"),
        compiler_params=pltpu.CompilerParams(needs_layout_passes=False,
                                             use_tc_tiling_on_sc=False),
        scratch_types=[
            pltpu.VMEM((W1N,), jnp.float32),
            pltpu.VMEM((CG, G), jnp.int32),
            pltpu.VMEM((CG, G), jnp.int32),
            pltpu.VMEM((CR, EMB), jnp.float32),
            pltpu.VMEM((CR, EMB), jnp.float32),
            pltpu.VMEM((CG, G), jnp.float32),
            pltpu.VMEM((CG, G), jnp.float32),
            pltpu.VMEM((CG, G), jnp.float32),
            pltpu.VMEM((CG, G), jnp.float32),
            pltpu.SemaphoreType.DMA,
            pltpu.SemaphoreType.DMA,
            pltpu.SemaphoreType.DMA,
            pltpu.SemaphoreType.DMA,
            pltpu.SemaphoreType.DMA,
            pltpu.SemaphoreType.DMA,
        ],
    )
    return wrapped(idx2, w2, w1, val2)


BLK = 1024
DIN = FP * EMB


def _tc_body(x_ref, nv_ref, yf_ref, w1a_ref, w1b_ref, b1_ref, w2_ref, b2_ref,
             wfa_ref, wfb_ref, bf_ref, out_ref):
    z1 = jnp.dot(x_ref[...], w1a_ref[...], preferred_element_type=jnp.float32)
    z1 = z1 + jnp.dot(nv_ref[...], w1b_ref[...],
                      preferred_element_type=jnp.float32)
    h = jnp.maximum(z1 + b1_ref[...], 0.0)
    z2 = jnp.dot(h, w2_ref[...], preferred_element_type=jnp.float32)
    r2 = jnp.maximum(z2 + b2_ref[...], 0.0)
    logits = (jnp.dot(yf_ref[...], wfa_ref[...],
                      preferred_element_type=jnp.float32)
              + jnp.dot(r2, wfb_ref[...], preferred_element_type=jnp.float32)
              + bf_ref[...])
    m = jnp.max(logits, axis=-1, keepdims=True)
    e = jnp.exp(logits - m)
    out_ref[...] = e / jnp.sum(e, axis=-1, keepdims=True)


def _tc_mlp(x, nv, yf, w1a, w1b, b1, w2, b2, wfa, wfb, bfv):
    return pl.pallas_call(
        _tc_body,
        grid=(B // BLK,),
        in_specs=[
            pl.BlockSpec((BLK, DIN), lambda i: (i, 0)),
            pl.BlockSpec((BLK, NUM), lambda i: (i, 0)),
            pl.BlockSpec((BLK, FP), lambda i: (i, 0)),
            pl.BlockSpec((DIN, 256), lambda i: (0, 0)),
            pl.BlockSpec((NUM, 256), lambda i: (0, 0)),
            pl.BlockSpec((1, 256), lambda i: (0, 0)),
            pl.BlockSpec((256, 128), lambda i: (0, 0)),
            pl.BlockSpec((1, 128), lambda i: (0, 0)),
            pl.BlockSpec((FP, 2), lambda i: (0, 0)),
            pl.BlockSpec((128, 2), lambda i: (0, 0)),
            pl.BlockSpec((1, 2), lambda i: (0, 0)),
        ],
        out_specs=pl.BlockSpec((BLK, 2), lambda i: (i, 0)),
        out_shape=jax.ShapeDtypeStruct((B, 2), jnp.float32),
    )(x, nv, yf, w1a, w1b, b1, w2, b2, wfa, wfb, bfv)


def kernel(single_index, numerical_index, numerical_value, value,
           W1s, W1n, W2s, W2n,
           D1_w, D1_b, bn1_g, bn1_b, bn1_m, bn1_v,
           D2_w, D2_b, bn2_g, bn2_b, bn2_m, bn2_v,
           Wf, bf):
    zcol_i = jnp.zeros((B, 1), jnp.int32)
    zcol_f = jnp.zeros((B, 1), jnp.float32)
    cat_idx = jnp.concatenate(
        [single_index, numerical_index + VOCAB, zcol_i], axis=1)
    idx2 = cat_idx.reshape(ROWS, G)
    w2 = jnp.concatenate([W2s, W2n], axis=0)
    w1 = jnp.concatenate([W1s, W1n], axis=0)[:, 0]
    val2 = jnp.concatenate([value, zcol_f], axis=1).reshape(ROWS, G)

    emb, yf2 = _sc_gather(idx2, w2, w1, val2)
    x = emb.reshape(B, DIN)
    yf = yf2.reshape(B, FP)

    s1 = bn1_g * lax.rsqrt(bn1_v + EPS)
    t1 = bn1_b - bn1_m * s1
    w2p = D2_w * s1[:, None]
    b2p = (D2_b + t1 @ D2_w)[None, :]
    s2 = bn2_g * lax.rsqrt(bn2_v + EPS)
    t2 = bn2_b - bn2_m * s2
    wfa = jnp.concatenate([Wf[:FIELD], jnp.zeros((1, 2), jnp.float32)], axis=0)
    wfb = Wf[FIELD:] * s2[:, None]
    bfp = (bf + t2 @ Wf[FIELD:])[None, :]

    w1a = jnp.concatenate(
        [D1_w[:FIELD * EMB], jnp.zeros((EMB, 256), jnp.float32)], axis=0)
    w1b = D1_w[FIELD * EMB:]
    b1 = D1_b[None, :]

    return _tc_mlp(x, numerical_value, yf, w1a, w1b, b1, w2p, b2p,
                   wfa, wfb, bfp)


# speedup vs baseline: 33.8888x; 1.3156x over previous
import jax
import jax.numpy as jnp
import numpy as np
from jax import lax
from jax.experimental import pallas as pl
from jax.experimental.pallas import tpu as pltpu
from jax.experimental.pallas import tpu_sc as plsc

B = 16384
SINGLE = 26
NUM = 13
EMB = 16
VOCAB = 10000
FIELD = SINGLE + NUM
FP = FIELD + 1
EPS = 1e-3

NC = 2
NS = 16
NW = NC * NS
G = 128
CB = 32
CGR = CB * FP // G
CR = CB * FP
W1N = 2 * VOCAB

NSPLIT = 2
BSL = B // NSPLIT
RPW = BSL // NW
NCHUNK = RPW // CB
NPAIR = NCHUNK // 2

_RF = np.zeros((5, 32), np.int32)
for _q in range(5):
    for _i in range(16):
        _p = 16 * _q + _i
        _RF[_q, _i] = _p % FP
        _RF[_q, 16 + _i] = _p // FP


def _make_sc_body(soff):
    def _sc_body(sidx_hbm, nidx_hbm, val_hbm, w2_hbm, w1_hbm, rf_hbm,
                 emb_out, yf_out,
                 w1_v, w2_sp, rf_v, sidx_v0, sidx_v1, nidx_v0, nidx_v1,
                 val_v0, val_v1, idx_v0, idx_v1, rows_v0, rows_v1,
                 yf_v0, yf_v1, in_s0, in_s1, g_s0, g_s1, out_s0, out_s1):
        sid = lax.axis_index("s")
        wid = sid * NC + lax.axis_index("c")
        b_base = wid * RPW
        @pl.when(sid == 0)
        def _():
            pltpu.sync_copy(w2_hbm, w2_sp)
        pltpu.sync_copy(rf_hbm, rf_v)
        pltpu.sync_copy(w1_hbm, w1_v)
        plsc.subcore_barrier()

        def start_in(c, sl_sidx, sl_nidx, sl_val, sem):
            b0 = soff + b_base + c * CB
            pltpu.async_copy(sidx_hbm.at[pl.ds(b0, CB)], sl_sidx, sem)
            pltpu.async_copy(nidx_hbm.at[pl.ds(b0, CB)], sl_nidx, sem)
            pltpu.async_copy(val_hbm.at[pl.ds(b0, CB)], sl_val, sem)

        def wait_in(sl_sidx, sl_nidx, sl_val, sem):
            pltpu.make_async_copy(
                sidx_hbm.at[pl.ds(0, CB)], sl_sidx, sem).wait()
            pltpu.make_async_copy(
                nidx_hbm.at[pl.ds(0, CB)], sl_nidx, sem).wait()
            pltpu.make_async_copy(
                val_hbm.at[pl.ds(0, CB)], sl_val, sem).wait()

        def build(sl_sidx, sl_nidx, sl_val, sl_idx, sl_yf):
            def body(vdiv, carry):
                for q in range(5):
                    v = 5 * vdiv + q
                    fv = rf_v[q, pl.ds(0, 16)]
                    rv = rf_v[q, pl.ds(16, 16)] + 2 * vdiv
                    fs = jnp.minimum(fv, SINGLE - 1)
                    sv = plsc.load_gather(sl_sidx, [rv, fs])
                    fn = jnp.clip(fv - SINGLE, 0, NUM - 1)
                    nv = plsc.load_gather(sl_nidx, [rv, fn]) + VOCAB
                    isn = fv >= SINGLE
                    ispad = fv >= FIELD
                    idxv = jnp.where(ispad, 0, jnp.where(isn, nv, sv))
                    row = lax.shift_right_logical(v, 3)
                    col = 16 * (v & 7)
                    sl_idx[row, pl.ds(col, 16)] = idxv
                    fvv = jnp.minimum(fv, FIELD - 1)
                    vv = plsc.load_gather(sl_val, [rv, fvv])
                    w1v = plsc.load_gather(w1_v, [idxv])
                    yfv = jnp.where(ispad, 0.0, vv * w1v)
                    plsc.store_scatter(sl_yf, [rv, fv], yfv)
                return carry
            lax.fori_loop(0, CR // 80, body, 0)

        def fire_g(sl_idx, sl_rows, sem):
            for g in range(CGR):
                pltpu.async_copy(w2_sp.at[sl_idx.at[g]],
                                 sl_rows.at[pl.ds(g * G, G)], sem)

        def wait_g(sl_rows, sem):
            pltpu.make_async_copy(w2_sp.at[pl.ds(0, CR)], sl_rows,
                                  sem).wait()

        def start_out(c, sl_rows, sl_yf, sem):
            b0 = b_base + c * CB
            pltpu.async_copy(sl_rows, emb_out.at[pl.ds(b0 * FP, CR)], sem)
            pltpu.async_copy(sl_yf, yf_out.at[pl.ds(b0, CB)], sem)

        def wait_out(sl_rows, sl_yf, sem):
            pltpu.make_async_copy(sl_rows, emb_out.at[pl.ds(0, CR)],
                                  sem).wait()
            pltpu.make_async_copy(sl_yf, yf_out.at[pl.ds(0, CB)],
                                  sem).wait()

        slots = (
            (sidx_v0, nidx_v0, val_v0, idx_v0, rows_v0, yf_v0,
             in_s0, g_s0, out_s0),
            (sidx_v1, nidx_v1, val_v1, idx_v1, rows_v1, yf_v1,
             in_s1, g_s1, out_s1))

        start_in(0, sidx_v0, nidx_v0, val_v0, in_s0)
        start_in(1, sidx_v1, nidx_v1, val_v1, in_s1)

        def pair(j, carry):
            for s in range(2):
                (sl_sidx, sl_nidx, sl_val, sl_idx, sl_rows, sl_yf,
                 in_s, g_s, out_s) = slots[s]

                @pl.when(j > 0)
                def _():
                    wait_out(sl_rows, sl_yf, out_s)

                wait_in(sl_sidx, sl_nidx, sl_val, in_s)
                build(sl_sidx, sl_nidx, sl_val, sl_idx, sl_yf)
                fire_g(sl_idx, sl_rows, g_s)
            for s in range(2):
                (sl_sidx, sl_nidx, sl_val, sl_idx, sl_rows, sl_yf,
                 in_s, g_s, out_s) = slots[s]
                c = 2 * j + s
                wait_g(sl_rows, g_s)
                start_out(c, sl_rows, sl_yf, out_s)

                @pl.when(j < NPAIR - 1)
                def _():
                    start_in(c + 2, sl_sidx, sl_nidx, sl_val, in_s)
            return carry

        lax.fori_loop(0, NPAIR, pair, 0)
        wait_out(rows_v0, yf_v0, out_s0)
        wait_out(rows_v1, yf_v1, out_s1)

    return _sc_body


def _sc_gather(soff, sidx, nidx, val, w2, w1, rf):
    wrapped = pl.kernel(
        _make_sc_body(soff),
        out_type=(
            jax.ShapeDtypeStruct((BSL * FP, EMB), jnp.float32),
            jax.ShapeDtypeStruct((BSL, FP), jnp.float32),
        ),
        mesh=plsc.VectorSubcoreMesh(core_axis_name="c", subcore_axis_name="s"),
        compiler_params=pltpu.CompilerParams(needs_layout_passes=False,
                                             use_tc_tiling_on_sc=False),
        scratch_types=[
            pltpu.VMEM((W1N,), jnp.float32),
            pltpu.VMEM_SHARED((W1N, EMB), jnp.float32),
            pltpu.VMEM((5, 32), jnp.int32),
            pltpu.VMEM((CB, SINGLE), jnp.int32),
            pltpu.VMEM((CB, SINGLE), jnp.int32),
            pltpu.VMEM((CB, NUM), jnp.int32),
            pltpu.VMEM((CB, NUM), jnp.int32),
            pltpu.VMEM((CB, FIELD), jnp.float32),
            pltpu.VMEM((CB, FIELD), jnp.float32),
            pltpu.VMEM((CGR, G), jnp.int32),
            pltpu.VMEM((CGR, G), jnp.int32),
            pltpu.VMEM((CR, EMB), jnp.float32),
            pltpu.VMEM((CR, EMB), jnp.float32),
            pltpu.VMEM((CB, FP), jnp.float32),
            pltpu.VMEM((CB, FP), jnp.float32),
            pltpu.SemaphoreType.DMA,
            pltpu.SemaphoreType.DMA,
            pltpu.SemaphoreType.DMA,
            pltpu.SemaphoreType.DMA,
            pltpu.SemaphoreType.DMA,
            pltpu.SemaphoreType.DMA,
        ],
    )
    return wrapped(sidx, nidx, val, w2, w1, rf)


BLK = 1024
DIN = FP * EMB


def _tc_body(x_ref, nv_ref, yf_ref, w1a_ref, w1b_ref, d1b_ref,
             g1_ref, b1_ref, m1_ref, v1_ref, d2w_ref, d2b_ref,
             g2_ref, b2_ref, m2_ref, v2_ref, wfa_ref, wfb_ref, bf_ref,
             out_ref):
    z1 = jnp.dot(x_ref[...], w1a_ref[...],
                 preferred_element_type=jnp.float32)
    z1 = z1 + jnp.dot(nv_ref[...], w1b_ref[...],
                      preferred_element_type=jnp.float32)
    h = jnp.maximum(z1 + d1b_ref[...], 0.0)
    s1 = g1_ref[...] * lax.rsqrt(v1_ref[...] + EPS)
    h = (h - m1_ref[...]) * s1 + b1_ref[...]
    z2 = jnp.dot(h, d2w_ref[...], preferred_element_type=jnp.float32)
    r2 = jnp.maximum(z2 + d2b_ref[...], 0.0)
    s2 = g2_ref[...] * lax.rsqrt(v2_ref[...] + EPS)
    r2 = (r2 - m2_ref[...]) * s2 + b2_ref[...]
    logits = (jnp.dot(yf_ref[...], wfa_ref[...],
                      preferred_element_type=jnp.float32)
              + jnp.dot(r2, wfb_ref[...], preferred_element_type=jnp.float32)
              + bf_ref[...])
    m = jnp.max(logits, axis=-1, keepdims=True)
    e = jnp.exp(logits - m)
    out_ref[...] = e / jnp.sum(e, axis=-1, keepdims=True)


def _tc_mlp(s, x, nv_full, yf, w1a, w1b, d1b, g1, b1, m1, v1, d2w, d2b,
            g2, b2, m2, v2, wfa, wfb, bfv):
    base = s * (BSL // BLK)
    return pl.pallas_call(
        _tc_body,
        grid=(BSL // BLK,),
        in_specs=[
            pl.BlockSpec((BLK, DIN), lambda i: (i, 0)),
            pl.BlockSpec((BLK, NUM), lambda i: (i + base, 0)),
            pl.BlockSpec((BLK, FP), lambda i: (i, 0)),
            pl.BlockSpec((DIN, 256), lambda i: (0, 0)),
            pl.BlockSpec((NUM, 256), lambda i: (0, 0)),
            pl.BlockSpec((1, 256), lambda i: (0, 0)),
            pl.BlockSpec((1, 256), lambda i: (0, 0)),
            pl.BlockSpec((1, 256), lambda i: (0, 0)),
            pl.BlockSpec((1, 256), lambda i: (0, 0)),
            pl.BlockSpec((1, 256), lambda i: (0, 0)),
            pl.BlockSpec((256, 128), lambda i: (0, 0)),
            pl.BlockSpec((1, 128), lambda i: (0, 0)),
            pl.BlockSpec((1, 128), lambda i: (0, 0)),
            pl.BlockSpec((1, 128), lambda i: (0, 0)),
            pl.BlockSpec((1, 128), lambda i: (0, 0)),
            pl.BlockSpec((1, 128), lambda i: (0, 0)),
            pl.BlockSpec((FP, 2), lambda i: (0, 0)),
            pl.BlockSpec((128, 2), lambda i: (0, 0)),
            pl.BlockSpec((1, 2), lambda i: (0, 0)),
        ],
        out_specs=pl.BlockSpec((BLK, 2), lambda i: (i, 0)),
        out_shape=jax.ShapeDtypeStruct((BSL, 2), jnp.float32),
    )(x, nv_full, yf, w1a, w1b, d1b, g1, b1, m1, v1, d2w, d2b,
      g2, b2, m2, v2, wfa, wfb, bfv)


def kernel(single_index, numerical_index, numerical_value, value,
           W1s, W1n, W2s, W2n,
           D1_w, D1_b, bn1_g, bn1_b, bn1_m, bn1_v,
           D2_w, D2_b, bn2_g, bn2_b, bn2_m, bn2_v,
           Wf, bf):
    w2 = jnp.concatenate([W2s, W2n], axis=0)
    w1 = jnp.concatenate([W1s, W1n], axis=0)[:, 0]
    rf = jnp.asarray(_RF)
    wfa = Wf[:FP]
    wfb = Wf[FIELD:]
    w1a = jnp.concatenate(
        [D1_w[:FIELD * EMB], jnp.zeros((EMB, 256), jnp.float32)], axis=0)
    w1b = D1_w[FIELD * EMB:]

    outs = []
    for s in range(NSPLIT):
        emb, yf2 = _sc_gather(s * BSL, single_index, numerical_index,
                              value, w2, w1, rf)
        x = emb.reshape(BSL, DIN)
        outs.append(_tc_mlp(
            s, x, numerical_value, yf2, w1a, w1b, D1_b[None, :],
            bn1_g[None, :], bn1_b[None, :], bn1_m[None, :], bn1_v[None, :],
            D2_w, D2_b[None, :],
            bn2_g[None, :], bn2_b[None, :], bn2_m[None, :], bn2_v[None, :],
            wfa, wfb, bf[None, :]))
    return jnp.concatenate(outs, axis=0)


# speedup vs baseline: 48.8879x; 1.4426x over previous
import jax
import jax.numpy as jnp
from jax import lax
from jax.experimental import pallas as pl
from jax.experimental.pallas import tpu as pltpu
from jax.experimental.pallas import tpu_sc as plsc

B = 16384
SINGLE = 26
NUM = 13
EMB = 16
VOCAB = 10000
FIELD = SINGLE + NUM
FP = FIELD + 1
EPS = 1e-3

NC = 2
NS = 16
NW = NC * NS
G = 128
CG = 8
CR = CG * G
W1N = 2 * VOCAB

NSPLIT = 2
BSL = B // NSPLIT
ROWS_S = BSL * FP // G
PER_W_ROWS = ROWS_S // NW
NPAIR = PER_W_ROWS // CG // 2


def _sc_body(idx_hbm, w2_hbm, w1_hbm, val_hbm, emb_out, yf_out,
             w1_v, w2_sp, idx_v0, idx_v1, rows_v0, rows_v1, val_v0, val_v1,
             yf_v0, yf_v1, in_s0, in_s1, g_s0, g_s1, out_s0, out_s1):
    sid = lax.axis_index("s")
    wid = sid * NC + lax.axis_index("c")
    row_base = wid * PER_W_ROWS
    @pl.when(sid == 0)
    def _():
        pltpu.sync_copy(w2_hbm, w2_sp)
    pltpu.sync_copy(w1_hbm, w1_v)
    plsc.subcore_barrier()

    def start_in(c, sl_idx, sl_val, sem):
        ro = row_base + c * CG
        pltpu.async_copy(idx_hbm.at[pl.ds(ro, CG)], sl_idx, sem)
        pltpu.async_copy(val_hbm.at[pl.ds(ro, CG)], sl_val, sem)

    def wait_in(sl_idx, sl_val, sem):
        pltpu.make_async_copy(idx_hbm.at[pl.ds(0, CG)], sl_idx, sem).wait()
        pltpu.make_async_copy(val_hbm.at[pl.ds(0, CG)], sl_val, sem).wait()

    def fire_g(sl_idx, sl_rows, sem):
        for g in range(CG):
            pltpu.async_copy(w2_sp.at[sl_idx.at[g]],
                             sl_rows.at[pl.ds(g * G, G)], sem)

    def wait_g(sl_rows, sem):
        pltpu.make_async_copy(w2_sp.at[pl.ds(0, CR)], sl_rows, sem).wait()

    def start_out(c, sl_rows, sl_yf, sem):
        ro = row_base + c * CG
        pltpu.async_copy(sl_rows, emb_out.at[pl.ds(ro * G, CR)], sem)
        pltpu.async_copy(sl_yf, yf_out.at[pl.ds(ro, CG)], sem)

    def wait_out(sl_rows, sl_yf, sem):
        pltpu.make_async_copy(sl_rows, emb_out.at[pl.ds(0, CR)], sem).wait()
        pltpu.make_async_copy(sl_yf, yf_out.at[pl.ds(0, CG)], sem).wait()

    def first_order(sl_idx, sl_val, sl_yf):
        def body(g, c):
            for k in range(G // 16):
                iv = sl_idx[g, pl.ds(k * 16, 16)]
                vals = plsc.load_gather(w1_v, [iv])
                sl_yf[g, pl.ds(k * 16, 16)] = (
                    vals * sl_val[g, pl.ds(k * 16, 16)])
            return c
        lax.fori_loop(0, CG, body, 0)

    slots = ((idx_v0, val_v0, rows_v0, yf_v0, in_s0, g_s0, out_s0),
             (idx_v1, val_v1, rows_v1, yf_v1, in_s1, g_s1, out_s1))

    start_in(0, idx_v0, val_v0, in_s0)
    start_in(1, idx_v1, val_v1, in_s1)

    def pair(j, carry):
        for s in range(2):
            sl_idx, sl_val, sl_rows, sl_yf, in_s, g_s, out_s = slots[s]

            @pl.when(j > 0)
            def _():
                wait_out(sl_rows, sl_yf, out_s)

            wait_in(sl_idx, sl_val, in_s)
            fire_g(sl_idx, sl_rows, g_s)
            first_order(sl_idx, sl_val, sl_yf)
        for s in range(2):
            sl_idx, sl_val, sl_rows, sl_yf, in_s, g_s, out_s = slots[s]
            c = 2 * j + s
            wait_g(sl_rows, g_s)
            start_out(c, sl_rows, sl_yf, out_s)

            @pl.when(j < NPAIR - 1)
            def _():
                start_in(c + 2, sl_idx, sl_val, in_s)
        return carry

    lax.fori_loop(0, NPAIR, pair, 0)
    wait_out(rows_v0, yf_v0, out_s0)
    wait_out(rows_v1, yf_v1, out_s1)


def _sc_gather(idx2, w2, w1, val2):
    wrapped = pl.kernel(
        _sc_body,
        out_type=(
            jax.ShapeDtypeStruct((ROWS_S * G, EMB), jnp.float32),
            jax.ShapeDtypeStruct((ROWS_S, G), jnp.float32),
        ),
        mesh=plsc.VectorSubcoreMesh(core_axis_name="c", subcore_axis_name="s"),
        compiler_params=pltpu.CompilerParams(needs_layout_passes=False,
                                             use_tc_tiling_on_sc=False),
        scratch_types=[
            pltpu.VMEM((W1N,), jnp.float32),
            pltpu.VMEM_SHARED((W1N, EMB), jnp.float32),
            pltpu.VMEM((CG, G), jnp.int32),
            pltpu.VMEM((CG, G), jnp.int32),
            pltpu.VMEM((CR, EMB), jnp.float32),
            pltpu.VMEM((CR, EMB), jnp.float32),
            pltpu.VMEM((CG, G), jnp.float32),
            pltpu.VMEM((CG, G), jnp.float32),
            pltpu.VMEM((CG, G), jnp.float32),
            pltpu.VMEM((CG, G), jnp.float32),
            pltpu.SemaphoreType.DMA,
            pltpu.SemaphoreType.DMA,
            pltpu.SemaphoreType.DMA,
            pltpu.SemaphoreType.DMA,
            pltpu.SemaphoreType.DMA,
            pltpu.SemaphoreType.DMA,
        ],
    )
    return wrapped(idx2, w2, w1, val2)


BLK = 1024
DIN = FP * EMB


def _tc_body(x_ref, nv_ref, yf_ref, w1a_ref, w1b_ref, d1b_ref,
             g1_ref, b1_ref, m1_ref, v1_ref, d2w_ref, d2b_ref,
             g2_ref, b2_ref, m2_ref, v2_ref, wfa_ref, wfb_ref, bf_ref,
             out_ref):
    x = x_ref[...].reshape(BLK, DIN)
    z1 = jnp.dot(x, w1a_ref[...],
                 preferred_element_type=jnp.float32)
    z1 = z1 + jnp.dot(nv_ref[...], w1b_ref[...],
                      preferred_element_type=jnp.float32)
    h = jnp.maximum(z1 + d1b_ref[...], 0.0)
    s1 = g1_ref[...] * lax.rsqrt(v1_ref[...] + EPS)
    h = (h - m1_ref[...]) * s1 + b1_ref[...]
    z2 = jnp.dot(h, d2w_ref[...], preferred_element_type=jnp.float32)
    r2 = jnp.maximum(z2 + d2b_ref[...], 0.0)
    s2 = g2_ref[...] * lax.rsqrt(v2_ref[...] + EPS)
    r2 = (r2 - m2_ref[...]) * s2 + b2_ref[...]
    logits = (jnp.dot(yf_ref[...], wfa_ref[...],
                      preferred_element_type=jnp.float32)
              + jnp.dot(r2, wfb_ref[...], preferred_element_type=jnp.float32)
              + bf_ref[...])
    m = jnp.max(logits, axis=-1, keepdims=True)
    e = jnp.exp(logits - m)
    out_ref[...] = e / jnp.sum(e, axis=-1, keepdims=True)


def _tc_mlp(s, x, nv_full, yf, w1a, w1b, d1b, g1, b1, m1, v1, d2w, d2b,
            g2, b2, m2, v2, wfa, wfb, bfv):
    base = s * (BSL // BLK)
    return pl.pallas_call(
        _tc_body,
        grid=(BSL // BLK,),
        in_specs=[
            pl.BlockSpec((BLK * DIN // G, G), lambda i: (i, 0)),
            pl.BlockSpec((BLK, NUM), lambda i: (i + base, 0)),
            pl.BlockSpec((BLK, FP), lambda i: (i, 0)),
            pl.BlockSpec((DIN, 256), lambda i: (0, 0)),
            pl.BlockSpec((NUM, 256), lambda i: (0, 0)),
            pl.BlockSpec((1, 256), lambda i: (0, 0)),
            pl.BlockSpec((1, 256), lambda i: (0, 0)),
            pl.BlockSpec((1, 256), lambda i: (0, 0)),
            pl.BlockSpec((1, 256), lambda i: (0, 0)),
            pl.BlockSpec((1, 256), lambda i: (0, 0)),
            pl.BlockSpec((256, 128), lambda i: (0, 0)),
            pl.BlockSpec((1, 128), lambda i: (0, 0)),
            pl.BlockSpec((1, 128), lambda i: (0, 0)),
            pl.BlockSpec((1, 128), lambda i: (0, 0)),
            pl.BlockSpec((1, 128), lambda i: (0, 0)),
            pl.BlockSpec((1, 128), lambda i: (0, 0)),
            pl.BlockSpec((FP, 2), lambda i: (0, 0)),
            pl.BlockSpec((128, 2), lambda i: (0, 0)),
            pl.BlockSpec((1, 2), lambda i: (0, 0)),
        ],
        out_specs=pl.BlockSpec((BLK, 2), lambda i: (i, 0)),
        out_shape=jax.ShapeDtypeStruct((BSL, 2), jnp.float32),
    )(x, nv_full, yf, w1a, w1b, d1b, g1, b1, m1, v1, d2w, d2b,
      g2, b2, m2, v2, wfa, wfb, bfv)


def kernel(single_index, numerical_index, numerical_value, value,
           W1s, W1n, W2s, W2n,
           D1_w, D1_b, bn1_g, bn1_b, bn1_m, bn1_v,
           D2_w, D2_b, bn2_g, bn2_b, bn2_m, bn2_v,
           Wf, bf):
    w2 = jnp.concatenate([W2s, W2n], axis=0)
    w1 = jnp.concatenate([W1s, W1n], axis=0)[:, 0]
    wfa = Wf[:FP]
    wfb = Wf[FIELD:]
    w1a = jnp.concatenate(
        [D1_w[:FIELD * EMB], jnp.zeros((EMB, 256), jnp.float32)], axis=0)
    w1b = D1_w[FIELD * EMB:]

    zcol_i = jnp.zeros((BSL, 1), jnp.int32)
    zcol_f = jnp.zeros((BSL, 1), jnp.float32)
    outs = []
    for s in range(NSPLIT):
        sl = slice(s * BSL, (s + 1) * BSL)
        idx2 = jnp.concatenate(
            [single_index[sl], numerical_index[sl] + VOCAB, zcol_i],
            axis=1).reshape(ROWS_S, G)
        val2 = jnp.concatenate(
            [value[sl], zcol_f], axis=1).reshape(ROWS_S, G)
        emb, yf2 = _sc_gather(idx2, w2, w1, val2)
        x = emb.reshape(BSL * DIN // G, G)
        yf = yf2.reshape(BSL, FP)
        outs.append(_tc_mlp(
            s, x, numerical_value, yf, w1a, w1b, D1_b[None, :],
            bn1_g[None, :], bn1_b[None, :], bn1_m[None, :], bn1_v[None, :],
            D2_w, D2_b[None, :],
            bn2_g[None, :], bn2_b[None, :], bn2_m[None, :], bn2_v[None, :],
            wfa, wfb, bf[None, :]))
    return jnp.concatenate(outs, axis=0)


# speedup vs baseline: 50.8333x; 1.0398x over previous
import jax
import jax.numpy as jnp
from jax import lax
from jax.experimental import pallas as pl
from jax.experimental.pallas import tpu as pltpu
from jax.experimental.pallas import tpu_sc as plsc

B = 16384
SINGLE = 26
NUM = 13
EMB = 16
VOCAB = 10000
FIELD = SINGLE + NUM
FP = FIELD + 1
EPS = 1e-3

NC = 2
NS = 16
NW = NC * NS
G = 128
CG = 10
CB = 32
CR = CG * G
W1N = 2 * VOCAB

import numpy as np
_RF = np.zeros((5, 32), np.int32)
for _q in range(5):
    for _i in range(16):
        _p = 16 * _q + _i
        _RF[_q, _i] = _p % FP
        _RF[_q, 16 + _i] = _p // FP

NSPLIT = 2
BSL = B // NSPLIT
ROWS_S = BSL * FP // G
PER_W_ROWS = ROWS_S // NW
NPAIR = PER_W_ROWS // CG // 2


def _sc_body(idx_hbm, w2_hbm, w1_hbm, val_hbm, rf_hbm, emb_out, yf_out,
             w1_v, w2_sp, rf_v, idx_v0, idx_v1, rows_v0, rows_v1,
             val_v0, val_v1, yf_v0, yf_v1,
             in_s0, in_s1, g_s0, g_s1, out_s0, out_s1):
    sid = lax.axis_index("s")
    wid = sid * NC + lax.axis_index("c")
    row_base = wid * PER_W_ROWS
    b_base = wid * (PER_W_ROWS * G // FP)
    @pl.when(sid == 0)
    def _():
        pltpu.sync_copy(w2_hbm, w2_sp)
    pltpu.sync_copy(rf_hbm, rf_v)
    pltpu.sync_copy(w1_hbm, w1_v)
    plsc.subcore_barrier()

    zvec = jnp.zeros((16,), jnp.float32)

    def zero_buf(buf):
        def zb(i, c):
            for k in range(G // 16):
                buf[i, pl.ds(16 * k, 16)] = zvec
            return c
        lax.fori_loop(0, CB, zb, 0)

    zero_buf(yf_v0)
    zero_buf(yf_v1)

    def start_in(c, sl_idx, sl_val, sem):
        ro = row_base + c * CG
        pltpu.async_copy(idx_hbm.at[pl.ds(ro, CG)], sl_idx, sem)
        pltpu.async_copy(val_hbm.at[pl.ds(ro, CG)], sl_val, sem)

    def wait_in(sl_idx, sl_val, sem):
        pltpu.make_async_copy(idx_hbm.at[pl.ds(0, CG)], sl_idx, sem).wait()
        pltpu.make_async_copy(val_hbm.at[pl.ds(0, CG)], sl_val, sem).wait()

    def fire_g(sl_idx, sl_rows, sem):
        for g in range(CG):
            pltpu.async_copy(w2_sp.at[sl_idx.at[g]],
                             sl_rows.at[pl.ds(g * G, G)], sem)

    def wait_g(sl_rows, sem):
        pltpu.make_async_copy(w2_sp.at[pl.ds(0, CR)], sl_rows, sem).wait()

    def start_out(c, sl_rows, sl_yf, sem):
        ro = row_base + c * CG
        pltpu.async_copy(sl_rows, emb_out.at[pl.ds(ro * G, CR)], sem)
        pltpu.async_copy(sl_yf, yf_out.at[pl.ds(b_base + c * CB, CB)], sem)

    def wait_out(sl_rows, sl_yf, sem):
        pltpu.make_async_copy(sl_rows, emb_out.at[pl.ds(0, CR)], sem).wait()
        pltpu.make_async_copy(sl_yf, yf_out.at[pl.ds(0, CB)], sem).wait()

    def first_order(sl_idx, sl_val, sl_yf):
        def body(vdiv, c):
            for q in range(5):
                v = 5 * vdiv + q
                fv = rf_v[q, pl.ds(0, 16)]
                rv = rf_v[q, pl.ds(16, 16)] + 2 * vdiv
                row = lax.shift_right_logical(v, 3)
                col = 16 * (v & 7)
                iv = sl_idx[row, pl.ds(col, 16)]
                vals = plsc.load_gather(w1_v, [iv])
                vv = sl_val[row, pl.ds(col, 16)]
                plsc.store_scatter(sl_yf, [rv, fv], vals * vv)
            return c
        lax.fori_loop(0, CR // 80, body, 0)

    slots = ((idx_v0, val_v0, rows_v0, yf_v0, in_s0, g_s0, out_s0),
             (idx_v1, val_v1, rows_v1, yf_v1, in_s1, g_s1, out_s1))

    start_in(0, idx_v0, val_v0, in_s0)
    start_in(1, idx_v1, val_v1, in_s1)

    def pair(j, carry):
        for s in range(2):
            sl_idx, sl_val, sl_rows, sl_yf, in_s, g_s, out_s = slots[s]

            @pl.when(j > 0)
            def _():
                wait_out(sl_rows, sl_yf, out_s)

            wait_in(sl_idx, sl_val, in_s)
            fire_g(sl_idx, sl_rows, g_s)
            first_order(sl_idx, sl_val, sl_yf)
        for s in range(2):
            sl_idx, sl_val, sl_rows, sl_yf, in_s, g_s, out_s = slots[s]
            c = 2 * j + s
            wait_g(sl_rows, g_s)
            start_out(c, sl_rows, sl_yf, out_s)

            @pl.when(j < NPAIR - 1)
            def _():
                start_in(c + 2, sl_idx, sl_val, in_s)
        return carry

    lax.fori_loop(0, NPAIR, pair, 0)
    wait_out(rows_v0, yf_v0, out_s0)
    wait_out(rows_v1, yf_v1, out_s1)


def _sc_gather(idx2, w2, w1, val2, rf):
    wrapped = pl.kernel(
        _sc_body,
        out_type=(
            jax.ShapeDtypeStruct((ROWS_S * G, EMB), jnp.float32),
            jax.ShapeDtypeStruct((BSL, G), jnp.float32),
        ),
        mesh=plsc.VectorSubcoreMesh(core_axis_name="c", subcore_axis_name="s"),
        compiler_params=pltpu.CompilerParams(needs_layout_passes=False,
                                             use_tc_tiling_on_sc=False),
        scratch_types=[
            pltpu.VMEM((W1N,), jnp.float32),
            pltpu.VMEM_SHARED((W1N, EMB), jnp.float32),
            pltpu.VMEM((5, 32), jnp.int32),
            pltpu.VMEM((CG, G), jnp.int32),
            pltpu.VMEM((CG, G), jnp.int32),
            pltpu.VMEM((CR, EMB), jnp.float32),
            pltpu.VMEM((CR, EMB), jnp.float32),
            pltpu.VMEM((CG, G), jnp.float32),
            pltpu.VMEM((CG, G), jnp.float32),
            pltpu.VMEM((CB, G), jnp.float32),
            pltpu.VMEM((CB, G), jnp.float32),
            pltpu.SemaphoreType.DMA,
            pltpu.SemaphoreType.DMA,
            pltpu.SemaphoreType.DMA,
            pltpu.SemaphoreType.DMA,
            pltpu.SemaphoreType.DMA,
            pltpu.SemaphoreType.DMA,
        ],
    )
    return wrapped(idx2, w2, w1, val2, rf)


BLK = 2048
DIN = FP * EMB


def _tc_body(x_ref, nv_ref, yf_ref, w1a_ref, w1b_ref, d1b_ref,
             g1_ref, b1_ref, m1_ref, v1_ref, d2w_ref, d2b_ref,
             g2_ref, b2_ref, m2_ref, v2_ref, wfa_ref, wfb_ref, bf_ref,
             out_ref):
    x = x_ref[...].reshape(BLK, DIN)
    z1 = jnp.dot(x, w1a_ref[...],
                 preferred_element_type=jnp.float32)
    z1 = z1 + jnp.dot(nv_ref[...], w1b_ref[...],
                      preferred_element_type=jnp.float32)
    h = jnp.maximum(z1 + d1b_ref[...], 0.0)
    s1 = g1_ref[...] * lax.rsqrt(v1_ref[...] + EPS)
    h = (h - m1_ref[...]) * s1 + b1_ref[...]
    z2 = jnp.dot(h, d2w_ref[...], preferred_element_type=jnp.float32)
    r2 = jnp.maximum(z2 + d2b_ref[...], 0.0)
    s2 = g2_ref[...] * lax.rsqrt(v2_ref[...] + EPS)
    r2 = (r2 - m2_ref[...]) * s2 + b2_ref[...]
    logits = (jnp.dot(yf_ref[...], wfa_ref[...],
                      preferred_element_type=jnp.float32)
              + jnp.dot(r2, wfb_ref[...], preferred_element_type=jnp.float32)
              + bf_ref[...])
    m = jnp.max(logits, axis=-1, keepdims=True)
    e = jnp.exp(logits - m)
    out_ref[...] = e / jnp.sum(e, axis=-1, keepdims=True)


def _tc_mlp(s, x, nv_full, yf, w1a, w1b, d1b, g1, b1, m1, v1, d2w, d2b,
            g2, b2, m2, v2, wfa, wfb, bfv):
    base = s * (BSL // BLK)
    return pl.pallas_call(
        _tc_body,
        grid=(BSL // BLK,),
        in_specs=[
            pl.BlockSpec((BLK * DIN // G, G), lambda i: (i, 0)),
            pl.BlockSpec((BLK, NUM), lambda i: (i + base, 0)),
            pl.BlockSpec((BLK, G), lambda i: (i, 0)),
            pl.BlockSpec((DIN, 256), lambda i: (0, 0)),
            pl.BlockSpec((NUM, 256), lambda i: (0, 0)),
            pl.BlockSpec((1, 256), lambda i: (0, 0)),
            pl.BlockSpec((1, 256), lambda i: (0, 0)),
            pl.BlockSpec((1, 256), lambda i: (0, 0)),
            pl.BlockSpec((1, 256), lambda i: (0, 0)),
            pl.BlockSpec((1, 256), lambda i: (0, 0)),
            pl.BlockSpec((256, 128), lambda i: (0, 0)),
            pl.BlockSpec((1, 128), lambda i: (0, 0)),
            pl.BlockSpec((1, 128), lambda i: (0, 0)),
            pl.BlockSpec((1, 128), lambda i: (0, 0)),
            pl.BlockSpec((1, 128), lambda i: (0, 0)),
            pl.BlockSpec((1, 128), lambda i: (0, 0)),
            pl.BlockSpec((G, 2), lambda i: (0, 0)),
            pl.BlockSpec((128, 2), lambda i: (0, 0)),
            pl.BlockSpec((1, 2), lambda i: (0, 0)),
        ],
        out_specs=pl.BlockSpec((BLK, 2), lambda i: (i, 0)),
        out_shape=jax.ShapeDtypeStruct((BSL, 2), jnp.float32),
    )(x, nv_full, yf, w1a, w1b, d1b, g1, b1, m1, v1, d2w, d2b,
      g2, b2, m2, v2, wfa, wfb, bfv)


def kernel(single_index, numerical_index, numerical_value, value,
           W1s, W1n, W2s, W2n,
           D1_w, D1_b, bn1_g, bn1_b, bn1_m, bn1_v,
           D2_w, D2_b, bn2_g, bn2_b, bn2_m, bn2_v,
           Wf, bf):
    w2 = jnp.concatenate([W2s, W2n], axis=0)
    w1 = jnp.concatenate([W1s, W1n], axis=0)[:, 0]
    wfa = jnp.concatenate(
        [Wf[:FP], jnp.zeros((G - FP, 2), jnp.float32)], axis=0)
    wfb = Wf[FIELD:]
    rf = jnp.asarray(_RF)
    w1a = jnp.concatenate(
        [D1_w[:FIELD * EMB], jnp.zeros((EMB, 256), jnp.float32)], axis=0)
    w1b = D1_w[FIELD * EMB:]

    zcol_i = jnp.zeros((BSL, 1), jnp.int32)
    zcol_f = jnp.zeros((BSL, 1), jnp.float32)
    outs = []
    for s in range(NSPLIT):
        sl = slice(s * BSL, (s + 1) * BSL)
        idx2 = jnp.concatenate(
            [single_index[sl], numerical_index[sl] + VOCAB, zcol_i],
            axis=1).reshape(ROWS_S, G)
        val2 = jnp.concatenate(
            [value[sl], zcol_f], axis=1).reshape(ROWS_S, G)
        emb, yf = _sc_gather(idx2, w2, w1, val2, rf)
        x = emb.reshape(BSL * DIN // G, G)
        outs.append(_tc_mlp(
            s, x, numerical_value, yf, w1a, w1b, D1_b[None, :],
            bn1_g[None, :], bn1_b[None, :], bn1_m[None, :], bn1_v[None, :],
            D2_w, D2_b[None, :],
            bn2_g[None, :], bn2_b[None, :], bn2_m[None, :], bn2_v[None, :],
            wfa, wfb, bf[None, :]))
    return jnp.concatenate(outs, axis=0)
